# baseline (speedup 1.0000x reference)
import functools

import jax
import jax.numpy as jnp
from jax import lax
from jax.experimental import pallas as pl
from jax.experimental.pallas import tpu as pltpu
from jax.experimental.pallas import tpu_sc as plsc

N = 10000
E = 320000
H = 64
W2 = 128
EPS = 1e-5
F32 = jnp.float32

NC = 2
NS = 16
NW = NC * NS
CH = 128
NP = 10240
EP = 327680
NCHUNK = EP // CH
CPW = NCHUNK // NW
CPW0 = 56
CPW1 = NCHUNK // NS - CPW0
HALF = NCHUNK // NC
CPS = HALF // NS
RPT = NP // NS
CW = 16

_mesh = plsc.VectorSubcoreMesh(core_axis_name="c", subcore_axis_name="s")


def _sds(shape):
    return jax.ShapeDtypeStruct(shape, F32)


@functools.partial(
    pl.kernel,
    out_type=_sds((EP, W2)),
    mesh=_mesh,
    scratch_types=[
        pltpu.VMEM((CPW1, CH), jnp.int32),
        pltpu.VMEM((CPW1, CH), jnp.int32),
        pltpu.VMEM((CH, H), F32),
        pltpu.VMEM((CH, H), F32),
        pltpu.VMEM((CH, H), F32),
        pltpu.VMEM((CH, H), F32),
        pltpu.SemaphoreType.DMA,
        pltpu.SemaphoreType.DMA,
        pltpu.SemaphoreType.DMA,
        pltpu.SemaphoreType.DMA,
    ],
    compiler_params=pltpu.CompilerParams(use_tc_tiling_on_sc=False),
)
def _sc_gather(a_hbm, b_hbm, src_hbm, dst_hbm, out_w,
               idxs, idxd, buf_d0, buf_d1, buf_s0, buf_s1,
               sg0, sg1, sw0, sw1):
    c = lax.axis_index("c")
    s = lax.axis_index("s")

    bufs_d = (buf_d0, buf_d1)
    bufs_s = (buf_s0, buf_s1)
    gsems = (sg0, sg1)
    wsems = (sw0, sw1)

    def fire_g(j, q):
        pltpu.async_copy(a_hbm.at[idxd.at[j]], bufs_d[q], gsems[q])
        pltpu.async_copy(b_hbm.at[idxs.at[j]], bufs_s[q], gsems[q])

    def wait_g(q):
        pltpu.make_async_copy(a_hbm.at[idxd.at[0]], bufs_d[q], gsems[q]).wait()
        pltpu.make_async_copy(b_hbm.at[idxs.at[0]], bufs_s[q], gsems[q]).wait()

    def fire_w(base, j, q):
        off = (base + j) * CH
        pltpu.async_copy(bufs_d[q], out_w.at[pl.ds(off, CH), pl.ds(0, H)],
                         wsems[q])
        pltpu.async_copy(bufs_s[q], out_w.at[pl.ds(off, CH), pl.ds(H, H)],
                         wsems[q])

    def wait_w(q):
        pltpu.make_async_copy(bufs_d[q], out_w.at[pl.ds(0, CH), pl.ds(0, H)],
                              wsems[q]).wait()
        pltpu.make_async_copy(bufs_s[q], out_w.at[pl.ds(0, CH), pl.ds(H, H)],
                              wsems[q]).wait()

    def run(base, n):
        pltpu.sync_copy(dst_hbm.at[pl.ds(base, n)], idxd.at[pl.ds(0, n)])
        pltpu.sync_copy(src_hbm.at[pl.ds(base, n)], idxs.at[pl.ds(0, n)])
        KD = 2
        K = n // KD
        for q in range(KD):
            fire_g(q, q)

        def body(t, carry):
            j0 = t * KD
            for q in range(KD):
                wait_g(q)
                fire_w(base, j0 + q, q)
            for q in range(KD):
                @pl.when(t < K - 1)
                def _():
                    wait_w(q)
                    fire_g(j0 + KD + q, q)
            return carry

        lax.fori_loop(0, K, body, 0)
        for q in range(KD):
            wait_w(q)

    @pl.when(c == 0)
    def _():
        run(s * CPW0, CPW0)

    @pl.when(c == 1)
    def _():
        run(NS * CPW0 + s * CPW1, CPW1)


@functools.partial(
    pl.kernel,
    out_type=_sds((NC * NP, W2)),
    mesh=_mesh,
    scratch_types=[
        pltpu.VMEM((CPS, CH), jnp.int32),
        pltpu.VMEM((CH, W2), F32),
        pltpu.VMEM((CH, W2), F32),
        pltpu.VMEM_SHARED((NP, W2), F32),
        pltpu.SemaphoreType.DMA,
        pltpu.SemaphoreType.DMA,
        pltpu.SemaphoreType.DMA,
        pltpu.SemaphoreType.DMA,
    ],
    compiler_params=pltpu.CompilerParams(use_tc_tiling_on_sc=False),
)
def _sc_scatter(m2_hbm, dst_hbm, out_p, idxd,
                rows0, rows1, acc,
                sr0, sr1, ss0, ss1):
    c = lax.axis_index("c")
    s = lax.axis_index("s")
    base = c * HALF + s * CPS
    pltpu.sync_copy(dst_hbm.at[pl.ds(base, CPS)], idxd)

    zv = jnp.zeros((16,), F32)

    def zbody(i, carry):
        for k in range(W2 // 16):
            rows0[i, pl.ds(k * 16, 16)] = zv
        return carry

    lax.fori_loop(0, CH, zbody, 0)
    for t in range(RPT // CH):
        pltpu.sync_copy(rows0, acc.at[pl.ds(s * RPT + t * CH, CH)])
    plsc.subcore_barrier()

    rows = (rows0, rows1)
    rsems = (sr0, sr1)
    ssems = (ss0, ss1)

    def fire_r(j, q):
        off = (base + j) * CH
        pltpu.async_copy(m2_hbm.at[pl.ds(off, CH)], rows[q], rsems[q])

    def wait_r(q):
        pltpu.make_async_copy(m2_hbm.at[pl.ds(0, CH)], rows[q], rsems[q]).wait()

    def fire_s(j, q):
        pltpu.async_copy(rows[q], acc.at[idxd.at[j]], ssems[q], add=True)

    def wait_s(q):
        pltpu.make_async_copy(rows[q], acc.at[idxd.at[0]], ssems[q]).wait()

    KD = 2
    K = CPS // KD
    for q in range(KD):
        fire_r(q, q)

    def body(t, carry):
        j0 = t * KD
        for q in range(KD):
            wait_r(q)
            fire_s(j0 + q, q)
        for q in range(KD):
            @pl.when(t < K - 1)
            def _():
                wait_s(q)
                fire_r(j0 + KD + q, q)
        return carry

    lax.fori_loop(0, K, body, 0)
    for q in range(KD):
        wait_s(q)

    plsc.subcore_barrier()
    row = c * NP + s * RPT
    pltpu.sync_copy(acc.at[pl.ds(s * RPT, RPT)], out_p.at[pl.ds(row, RPT)])


@functools.partial(
    pl.kernel,
    out_type=_sds((NC * NP, CW)),
    mesh=_mesh,
    scratch_types=[
        pltpu.VMEM((CPS, CH), jnp.int32),
        pltpu.VMEM((CH, CW), F32),
        pltpu.VMEM_SHARED((NP, CW), F32),
        pltpu.SemaphoreType.DMA,
    ],
    compiler_params=pltpu.CompilerParams(use_tc_tiling_on_sc=False),
)
def _sc_count(dst_hbm, zeros_hbm, ones_hbm, out_c, idxd, ones_v, cnt, sc0):
    c = lax.axis_index("c")
    s = lax.axis_index("s")
    base = c * HALF + s * CPS
    pltpu.sync_copy(dst_hbm.at[pl.ds(base, CPS)], idxd)
    pltpu.sync_copy(zeros_hbm.at[pl.ds(s * RPT, RPT)], cnt.at[pl.ds(s * RPT, RPT)])
    pltpu.sync_copy(ones_hbm, ones_v)
    plsc.subcore_barrier()

    def fire_c(j, carry):
        pltpu.async_copy(ones_v, cnt.at[idxd.at[j]], sc0, add=True)
        return carry

    def drain_c(j, carry):
        pltpu.make_async_copy(ones_v, cnt.at[idxd.at[0]], sc0).wait()
        return carry

    lax.fori_loop(0, CPS, fire_c, 0)
    lax.fori_loop(0, CPS, drain_c, 0)

    plsc.subcore_barrier()
    row = c * NP + s * RPT
    pltpu.sync_copy(cnt.at[pl.ds(s * RPT, RPT)], out_c.at[pl.ds(row, RPT)])


def _pre_body(x_ref, wi_ref, wj_ref, b_ref, a_ref, b_out_ref):
    xv = x_ref[...]
    a_ref[...] = jnp.dot(xv, wi_ref[...], preferred_element_type=F32)
    b_out_ref[...] = jnp.dot(xv, wj_ref[...], preferred_element_type=F32) + b_ref[...]


def _tc_pre(x2, wi_t, wj_t, b1):
    return pl.pallas_call(
        _pre_body,
        out_shape=(_sds((NP, H)), _sds((NP, H))),
    )(x2, wi_t, wj_t, b1.reshape(1, H))


BE = 4096


def _edge_body(g_ref, w_ref, b_ref, o_ref):
    h = jnp.maximum(g_ref[:, :H] + g_ref[:, H:], 0.0)
    m2 = jnp.maximum(
        jnp.dot(h, w_ref[...], preferred_element_type=F32) + b_ref[...], 0.0)
    o_ref[...] = jnp.concatenate([m2, jnp.zeros((BE, H), F32)], axis=1)


def _tc_edge(gw, w2_t, b2):
    return pl.pallas_call(
        _edge_body,
        grid=(EP // BE,),
        in_specs=[
            pl.BlockSpec((BE, W2), lambda i: (i, 0)),
            pl.BlockSpec((H, H), lambda i: (0, 0)),
            pl.BlockSpec((1, H), lambda i: (0, 0)),
        ],
        out_specs=pl.BlockSpec((BE, W2), lambda i: (i, 0)),
        out_shape=_sds((EP, W2)),
    )(gw, w2_t, b2.reshape(1, H))


def _make_node_body(res, has_next):
    def body(*refs):
        if has_next:
            (x_ref, p_ref, c_ref, w1x_ref, w1a_ref, b1_ref, w2_ref, b2_ref,
             wi_ref, wj_ref, bn_ref, xo_ref, ao_ref, bo_ref) = refs
        else:
            (x_ref, p_ref, c_ref, w1x_ref, w1a_ref, b1_ref, w2_ref, b2_ref,
             xo_ref) = refs
        xv = x_ref[...]
        ssum = (p_ref[0] + p_ref[1])[:, :H]
        cnt = jnp.maximum(c_ref[0] + c_ref[1], 1.0)[:, :1]
        agg = ssum / cnt
        u = jnp.maximum(
            jnp.dot(xv, w1x_ref[...], preferred_element_type=F32)
            + jnp.dot(agg, w1a_ref[...], preferred_element_type=F32)
            + b1_ref[...], 0.0)
        v = jnp.maximum(
            jnp.dot(u, w2_ref[...], preferred_element_type=F32) + b2_ref[...], 0.0)
        if res:
            v = v + xv
        mask = lax.broadcasted_iota(jnp.int32, (NP, 1), 0) < N
        vm = jnp.where(mask, v, 0.0)
        mu = jnp.sum(vm, axis=0, keepdims=True) * (1.0 / N)
        d = v - mu
        dm = jnp.where(mask, d, 0.0)
        var = jnp.sum(dm * dm, axis=0, keepdims=True) * (1.0 / N)
        xn = d / jnp.sqrt(var + EPS)
        xo_ref[...] = xn
        if has_next:
            ao_ref[...] = jnp.dot(xn, wi_ref[...], preferred_element_type=F32)
            bo_ref[...] = jnp.dot(xn, wj_ref[...], preferred_element_type=F32) + bn_ref[...]
    return body


def _tc_node(x2, p, cnt16, w1x_t, w1a_t, b1, w2_t, b2, cdim, res, nxt):
    if nxt is None:
        body = _make_node_body(res, False)
        return pl.pallas_call(body, out_shape=_sds((NP, cdim)))(
            x2, p, cnt16, w1x_t, w1a_t, b1.reshape(1, H), w2_t,
            b2.reshape(1, cdim))
    wi_t, wj_t, bn = nxt
    body = _make_node_body(res, True)
    return pl.pallas_call(
        body, out_shape=(_sds((NP, cdim)), _sds((NP, H)), _sds((NP, H))),
    )(x2, p, cnt16, w1x_t, w1a_t, b1.reshape(1, H), w2_t, b2.reshape(1, cdim),
      wi_t, wj_t, bn.reshape(1, H))


_DIMS = [(128, 64, False), (64, 64, True), (64, 128, False)]


def kernel(x, edge_index,
           We1_0, be1_0, We2_0, be2_0, Wn1_0, bn1_0, Wn2_0, bn2_0,
           We1_1, be1_1, We2_1, be2_1, Wn1_1, bn1_1, Wn2_1, bn2_1,
           We1_2, be1_2, We2_2, be2_2, Wn1_2, bn1_2, Wn2_2, bn2_2):
    params = [
        (We1_0, be1_0, We2_0, be2_0, Wn1_0, bn1_0, Wn2_0, bn2_0),
        (We1_1, be1_1, We2_1, be2_1, Wn1_1, bn1_1, Wn2_1, bn2_1),
        (We1_2, be1_2, We2_2, be2_2, Wn1_2, bn1_2, Wn2_2, bn2_2),
    ]
    x2 = jnp.zeros((NP, _DIMS[0][0]), dtype=F32).at[:N].set(x[0])
    pad_src = jnp.zeros((EP - E,), dtype=jnp.int32)
    pad_dst = jnp.full((EP - E,), N, dtype=jnp.int32)
    src_r = jnp.concatenate([edge_index[0], pad_src]).reshape(NCHUNK, CH)
    dst_r = jnp.concatenate([edge_index[1], pad_dst]).reshape(NCHUNK, CH)
    zeros16 = jnp.zeros((NP, CW), dtype=F32)
    ones = jnp.ones((CH, CW), dtype=F32)

    wsplit = []
    for li, (f, cdim, res) in enumerate(_DIMS):
        We1, be1, We2, be2, Wn1, bn1, Wn2, bn2 = params[li]
        wsplit.append(dict(
            wi_t=We1[:, :f].T, wj_t=We1[:, f:].T, be1=be1,
            w2_t=We2.T, be2=be2,
            w1x_t=Wn1[:, :f].T, w1a_t=Wn1[:, f:].T, bn1=bn1,
            wn2_t=Wn2.T, bn2=bn2, cdim=cdim, res=res,
        ))

    a_nodes, b_nodes = _tc_pre(x2, wsplit[0]["wi_t"], wsplit[0]["wj_t"],
                               wsplit[0]["be1"])
    cnt16 = _sc_count(dst_r, zeros16, ones).reshape(NC, NP, CW)
    for li in range(3):
        w = wsplit[li]
        gw = _sc_gather(a_nodes, b_nodes, src_r, dst_r)
        m2 = _tc_edge(gw, w["w2_t"], w["be2"])
        p = _sc_scatter(m2, dst_r).reshape(NC, NP, W2)
        if li < 2:
            nxt = (wsplit[li + 1]["wi_t"], wsplit[li + 1]["wj_t"],
                   wsplit[li + 1]["be1"])
            x2, a_nodes, b_nodes = _tc_node(
                x2, p, cnt16, w["w1x_t"], w["w1a_t"], w["bn1"], w["wn2_t"],
                w["bn2"], w["cdim"], w["res"], nxt)
        else:
            x2 = _tc_node(x2, p, cnt16, w["w1x_t"], w["w1a_t"], w["bn1"],
                          w["wn2_t"], w["bn2"], w["cdim"], w["res"], None)
    return x2[:N].reshape(1, N, _DIMS[2][1])


# speedup vs baseline: 1.0157x; 1.0157x over previous
import functools

import jax
import jax.numpy as jnp
from jax import lax
from jax.experimental import pallas as pl
from jax.experimental.pallas import tpu as pltpu
from jax.experimental.pallas import tpu_sc as plsc

N = 10000
E = 320000
H = 64
W2 = 128
EPS = 1e-5
F32 = jnp.float32

NC = 2
NS = 16
NW = NC * NS
CH = 128
NP = 10240
EP = 327680
NCHUNK = EP // CH
CPW = NCHUNK // NW
CPW0 = 112
CPW1 = NCHUNK // NS - CPW0
CPWMAX = max(CPW0, CPW1)
HALF = NCHUNK // NC
CPS = HALF // NS
RPT = NP // NS
CW = 16

_mesh = plsc.VectorSubcoreMesh(core_axis_name="c", subcore_axis_name="s")


def _sds(shape):
    return jax.ShapeDtypeStruct(shape, F32)


@functools.partial(
    pl.kernel,
    out_type=_sds((EP, W2)),
    mesh=_mesh,
    scratch_types=[
        pltpu.VMEM((CPWMAX, CH), jnp.int32),
        pltpu.VMEM((CPWMAX, CH), jnp.int32),
        pltpu.VMEM((CH, H), F32),
        pltpu.VMEM((CH, H), F32),
        pltpu.VMEM((CH, H), F32),
        pltpu.VMEM((CH, H), F32),
        pltpu.SemaphoreType.DMA,
        pltpu.SemaphoreType.DMA,
        pltpu.SemaphoreType.DMA,
        pltpu.SemaphoreType.DMA,
    ],
    compiler_params=pltpu.CompilerParams(use_tc_tiling_on_sc=False),
)
def _sc_gather(a_hbm, b_hbm, src_hbm, dst_hbm, out_w,
               idxs, idxd, buf_d0, buf_d1, buf_s0, buf_s1,
               sg0, sg1, sw0, sw1):
    c = lax.axis_index("c")
    s = lax.axis_index("s")

    bufs_d = (buf_d0, buf_d1)
    bufs_s = (buf_s0, buf_s1)
    gsems = (sg0, sg1)
    wsems = (sw0, sw1)

    def fire_g(j, q):
        pltpu.async_copy(a_hbm.at[idxd.at[j]], bufs_d[q], gsems[q])
        pltpu.async_copy(b_hbm.at[idxs.at[j]], bufs_s[q], gsems[q])

    def wait_g(q):
        pltpu.make_async_copy(a_hbm.at[idxd.at[0]], bufs_d[q], gsems[q]).wait()
        pltpu.make_async_copy(b_hbm.at[idxs.at[0]], bufs_s[q], gsems[q]).wait()

    def fire_w(base, j, q):
        off = (base + j) * CH
        pltpu.async_copy(bufs_d[q], out_w.at[pl.ds(off, CH), pl.ds(0, H)],
                         wsems[q])
        pltpu.async_copy(bufs_s[q], out_w.at[pl.ds(off, CH), pl.ds(H, H)],
                         wsems[q])

    def wait_w(q):
        pltpu.make_async_copy(bufs_d[q], out_w.at[pl.ds(0, CH), pl.ds(0, H)],
                              wsems[q]).wait()
        pltpu.make_async_copy(bufs_s[q], out_w.at[pl.ds(0, CH), pl.ds(H, H)],
                              wsems[q]).wait()

    def run(base, n):
        pltpu.sync_copy(dst_hbm.at[pl.ds(base, n)], idxd.at[pl.ds(0, n)])
        pltpu.sync_copy(src_hbm.at[pl.ds(base, n)], idxs.at[pl.ds(0, n)])
        KD = 2
        K = n // KD
        for q in range(KD):
            fire_g(q, q)

        def body(t, carry):
            j0 = t * KD
            for q in range(KD):
                wait_g(q)
                fire_w(base, j0 + q, q)
            for q in range(KD):
                @pl.when(t < K - 1)
                def _():
                    wait_w(q)
                    fire_g(j0 + KD + q, q)
            return carry

        lax.fori_loop(0, K, body, 0)
        for q in range(KD):
            wait_w(q)

    @pl.when(c == 0)
    def _():
        run(s * CPW0, CPW0)

    @pl.when(c == 1)
    def _():
        run(NS * CPW0 + s * CPW1, CPW1)


@functools.partial(
    pl.kernel,
    out_type=_sds((NC * NP, W2)),
    mesh=_mesh,
    scratch_types=[
        pltpu.VMEM((CPS, CH), jnp.int32),
        pltpu.VMEM((CH, W2), F32),
        pltpu.VMEM((CH, W2), F32),
        pltpu.VMEM_SHARED((NP, W2), F32),
        pltpu.SemaphoreType.DMA,
        pltpu.SemaphoreType.DMA,
        pltpu.SemaphoreType.DMA,
        pltpu.SemaphoreType.DMA,
    ],
    compiler_params=pltpu.CompilerParams(use_tc_tiling_on_sc=False),
)
def _sc_scatter(m2_hbm, dst_hbm, out_p, idxd,
                rows0, rows1, acc,
                sr0, sr1, ss0, ss1):
    c = lax.axis_index("c")
    s = lax.axis_index("s")
    base = c * HALF + s * CPS
    pltpu.sync_copy(dst_hbm.at[pl.ds(base, CPS)], idxd)

    zv = jnp.zeros((16,), F32)

    def zbody(i, carry):
        for k in range(W2 // 16):
            rows0[i, pl.ds(k * 16, 16)] = zv
        return carry

    lax.fori_loop(0, CH, zbody, 0)
    for t in range(RPT // CH):
        pltpu.sync_copy(rows0, acc.at[pl.ds(s * RPT + t * CH, CH)])
    plsc.subcore_barrier()

    rows = (rows0, rows1)
    rsems = (sr0, sr1)
    ssems = (ss0, ss1)

    def fire_r(j, q):
        off = (base + j) * CH
        pltpu.async_copy(m2_hbm.at[pl.ds(off, CH)], rows[q], rsems[q])

    def wait_r(q):
        pltpu.make_async_copy(m2_hbm.at[pl.ds(0, CH)], rows[q], rsems[q]).wait()

    def fire_s(j, q):
        pltpu.async_copy(rows[q], acc.at[idxd.at[j]], ssems[q], add=True)

    def wait_s(q):
        pltpu.make_async_copy(rows[q], acc.at[idxd.at[0]], ssems[q]).wait()

    KD = 2
    K = CPS // KD
    for q in range(KD):
        fire_r(q, q)

    def body(t, carry):
        j0 = t * KD
        for q in range(KD):
            wait_r(q)
            fire_s(j0 + q, q)
        for q in range(KD):
            @pl.when(t < K - 1)
            def _():
                wait_s(q)
                fire_r(j0 + KD + q, q)
        return carry

    lax.fori_loop(0, K, body, 0)
    for q in range(KD):
        wait_s(q)

    plsc.subcore_barrier()
    row = c * NP + s * RPT
    pltpu.sync_copy(acc.at[pl.ds(s * RPT, RPT)], out_p.at[pl.ds(row, RPT)])


@functools.partial(
    pl.kernel,
    out_type=_sds((NC * NP, CW)),
    mesh=_mesh,
    scratch_types=[
        pltpu.VMEM((CPS, CH), jnp.int32),
        pltpu.VMEM((CH, CW), F32),
        pltpu.VMEM_SHARED((NP, CW), F32),
        pltpu.SemaphoreType.DMA,
    ],
    compiler_params=pltpu.CompilerParams(use_tc_tiling_on_sc=False),
)
def _sc_count(dst_hbm, zeros_hbm, ones_hbm, out_c, idxd, ones_v, cnt, sc0):
    c = lax.axis_index("c")
    s = lax.axis_index("s")
    base = c * HALF + s * CPS
    pltpu.sync_copy(dst_hbm.at[pl.ds(base, CPS)], idxd)
    pltpu.sync_copy(zeros_hbm.at[pl.ds(s * RPT, RPT)], cnt.at[pl.ds(s * RPT, RPT)])
    pltpu.sync_copy(ones_hbm, ones_v)
    plsc.subcore_barrier()

    def fire_c(j, carry):
        pltpu.async_copy(ones_v, cnt.at[idxd.at[j]], sc0, add=True)
        return carry

    def drain_c(j, carry):
        pltpu.make_async_copy(ones_v, cnt.at[idxd.at[0]], sc0).wait()
        return carry

    lax.fori_loop(0, CPS, fire_c, 0)
    lax.fori_loop(0, CPS, drain_c, 0)

    plsc.subcore_barrier()
    row = c * NP + s * RPT
    pltpu.sync_copy(cnt.at[pl.ds(s * RPT, RPT)], out_c.at[pl.ds(row, RPT)])


def _pre_body(x_ref, wi_ref, wj_ref, b_ref, a_ref, b_out_ref):
    xv = x_ref[...]
    a_ref[...] = jnp.dot(xv, wi_ref[...], preferred_element_type=F32)
    b_out_ref[...] = jnp.dot(xv, wj_ref[...], preferred_element_type=F32) + b_ref[...]


def _tc_pre(x2, wi_t, wj_t, b1):
    return pl.pallas_call(
        _pre_body,
        out_shape=(_sds((NP, H)), _sds((NP, H))),
    )(x2, wi_t, wj_t, b1.reshape(1, H))


BE = 4096


def _edge_body(g_ref, w_ref, b_ref, o_ref):
    h = jnp.maximum(g_ref[:, :H] + g_ref[:, H:], 0.0)
    m2 = jnp.maximum(
        jnp.dot(h, w_ref[...], preferred_element_type=F32) + b_ref[...], 0.0)
    o_ref[...] = jnp.concatenate([m2, jnp.zeros((BE, H), F32)], axis=1)


def _tc_edge(gw, w2_t, b2):
    return pl.pallas_call(
        _edge_body,
        grid=(EP // BE,),
        in_specs=[
            pl.BlockSpec((BE, W2), lambda i: (i, 0)),
            pl.BlockSpec((H, H), lambda i: (0, 0)),
            pl.BlockSpec((1, H), lambda i: (0, 0)),
        ],
        out_specs=pl.BlockSpec((BE, W2), lambda i: (i, 0)),
        out_shape=_sds((EP, W2)),
    )(gw, w2_t, b2.reshape(1, H))


def _make_node_body(res, has_next):
    def body(*refs):
        if has_next:
            (x_ref, p_ref, c_ref, w1x_ref, w1a_ref, b1_ref, w2_ref, b2_ref,
             wi_ref, wj_ref, bn_ref, xo_ref, ao_ref, bo_ref) = refs
        else:
            (x_ref, p_ref, c_ref, w1x_ref, w1a_ref, b1_ref, w2_ref, b2_ref,
             xo_ref) = refs
        xv = x_ref[...]
        ssum = (p_ref[0] + p_ref[1])[:, :H]
        cnt = jnp.maximum(c_ref[0] + c_ref[1], 1.0)[:, :1]
        agg = ssum / cnt
        u = jnp.maximum(
            jnp.dot(xv, w1x_ref[...], preferred_element_type=F32)
            + jnp.dot(agg, w1a_ref[...], preferred_element_type=F32)
            + b1_ref[...], 0.0)
        v = jnp.maximum(
            jnp.dot(u, w2_ref[...], preferred_element_type=F32) + b2_ref[...], 0.0)
        if res:
            v = v + xv
        mask = lax.broadcasted_iota(jnp.int32, (NP, 1), 0) < N
        vm = jnp.where(mask, v, 0.0)
        mu = jnp.sum(vm, axis=0, keepdims=True) * (1.0 / N)
        d = v - mu
        dm = jnp.where(mask, d, 0.0)
        var = jnp.sum(dm * dm, axis=0, keepdims=True) * (1.0 / N)
        xn = d / jnp.sqrt(var + EPS)
        xo_ref[...] = xn
        if has_next:
            ao_ref[...] = jnp.dot(xn, wi_ref[...], preferred_element_type=F32)
            bo_ref[...] = jnp.dot(xn, wj_ref[...], preferred_element_type=F32) + bn_ref[...]
    return body


def _tc_node(x2, p, cnt16, w1x_t, w1a_t, b1, w2_t, b2, cdim, res, nxt):
    if nxt is None:
        body = _make_node_body(res, False)
        return pl.pallas_call(body, out_shape=_sds((NP, cdim)))(
            x2, p, cnt16, w1x_t, w1a_t, b1.reshape(1, H), w2_t,
            b2.reshape(1, cdim))
    wi_t, wj_t, bn = nxt
    body = _make_node_body(res, True)
    return pl.pallas_call(
        body, out_shape=(_sds((NP, cdim)), _sds((NP, H)), _sds((NP, H))),
    )(x2, p, cnt16, w1x_t, w1a_t, b1.reshape(1, H), w2_t, b2.reshape(1, cdim),
      wi_t, wj_t, bn.reshape(1, H))


_DIMS = [(128, 64, False), (64, 64, True), (64, 128, False)]


def kernel(x, edge_index,
           We1_0, be1_0, We2_0, be2_0, Wn1_0, bn1_0, Wn2_0, bn2_0,
           We1_1, be1_1, We2_1, be2_1, Wn1_1, bn1_1, Wn2_1, bn2_1,
           We1_2, be1_2, We2_2, be2_2, Wn1_2, bn1_2, Wn2_2, bn2_2):
    params = [
        (We1_0, be1_0, We2_0, be2_0, Wn1_0, bn1_0, Wn2_0, bn2_0),
        (We1_1, be1_1, We2_1, be2_1, Wn1_1, bn1_1, Wn2_1, bn2_1),
        (We1_2, be1_2, We2_2, be2_2, Wn1_2, bn1_2, Wn2_2, bn2_2),
    ]
    x2 = jnp.zeros((NP, _DIMS[0][0]), dtype=F32).at[:N].set(x[0])
    pad_src = jnp.zeros((EP - E,), dtype=jnp.int32)
    pad_dst = jnp.full((EP - E,), N, dtype=jnp.int32)
    src_r = jnp.concatenate([edge_index[0], pad_src]).reshape(NCHUNK, CH)
    dst_r = jnp.concatenate([edge_index[1], pad_dst]).reshape(NCHUNK, CH)
    zeros16 = jnp.zeros((NP, CW), dtype=F32)
    ones = jnp.ones((CH, CW), dtype=F32)

    wsplit = []
    for li, (f, cdim, res) in enumerate(_DIMS):
        We1, be1, We2, be2, Wn1, bn1, Wn2, bn2 = params[li]
        wsplit.append(dict(
            wi_t=We1[:, :f].T, wj_t=We1[:, f:].T, be1=be1,
            w2_t=We2.T, be2=be2,
            w1x_t=Wn1[:, :f].T, w1a_t=Wn1[:, f:].T, bn1=bn1,
            wn2_t=Wn2.T, bn2=bn2, cdim=cdim, res=res,
        ))

    a_nodes, b_nodes = _tc_pre(x2, wsplit[0]["wi_t"], wsplit[0]["wj_t"],
                               wsplit[0]["be1"])
    cnt16 = _sc_count(dst_r, zeros16, ones).reshape(NC, NP, CW)
    for li in range(3):
        w = wsplit[li]
        gw = _sc_gather(a_nodes, b_nodes, src_r, dst_r)
        m2 = _tc_edge(gw, w["w2_t"], w["be2"])
        p = _sc_scatter(m2, dst_r).reshape(NC, NP, W2)
        if li < 2:
            nxt = (wsplit[li + 1]["wi_t"], wsplit[li + 1]["wj_t"],
                   wsplit[li + 1]["be1"])
            x2, a_nodes, b_nodes = _tc_node(
                x2, p, cnt16, w["w1x_t"], w["w1a_t"], w["bn1"], w["wn2_t"],
                w["bn2"], w["cdim"], w["res"], nxt)
        else:
            x2 = _tc_node(x2, p, cnt16, w["w1x_t"], w["w1a_t"], w["bn1"],
                          w["wn2_t"], w["bn2"], w["cdim"], w["res"], None)
    return x2[:N].reshape(1, N, _DIMS[2][1])
